# Initial kernel scaffold; baseline (speedup 1.0000x reference)
import jax
import jax.numpy as jnp
from jax.experimental import pallas as pl


def kernel(atom_features, bond_features, atom_neighbor_list, bond_neighbor_list, atom_mask, W_atom, b_atom, W_nei, b_nei, align_W, align_b, attend_W, attend_b, gru_Wih, gru_Whh, gru_bih, gru_bhh, mol_align_W, mol_align_b, mol_attend_W, mol_attend_b, mol_gru_Wih, mol_gru_Whh, mol_gru_bih, mol_gru_bhh, dnn_W1, dnn_b1, dnn_W2, dnn_b2, dnn_W3, dnn_b3):
    raise NotImplementedError("write your pallas kernel here")


# speedup vs baseline: 15.9812x; 15.9812x over previous
import jax
import jax.numpy as jnp
from jax import lax
from jax.experimental import pallas as pl
from jax.experimental.pallas import tpu as pltpu

B, A, D, M = 128, 64, 8, 96
ATOM_F, BOND_F, FP = 39, 10, 128
NEG = -9e8
MB = 8
R = MB * A
GRID = B // MB

f32 = jnp.float32


def _gru(x, h, WihT, WhhT, bih, bhh):
    gi = jnp.dot(x, WihT, preferred_element_type=f32) + bih
    gh = jnp.dot(h, WhhT, preferred_element_type=f32) + bhh
    r = jax.nn.sigmoid(gi[:, :FP] + gh[:, :FP])
    z = jax.nn.sigmoid(gi[:, FP:2 * FP] + gh[:, FP:2 * FP])
    n = jnp.tanh(gi[:, 2 * FP:] + r * gh[:, 2 * FP:])
    return (1.0 - z) * n + z * h


def _elu(x):
    xn = jnp.minimum(x, 0.0)
    return jnp.where(x > 0, x, jnp.exp(xn) - 1.0)


def _softmax_last(score):
    m = jnp.max(score, axis=-1, keepdims=True)
    e = jnp.exp(score - m)
    return e / jnp.sum(e, axis=-1, keepdims=True)


def _fused_kernel(af_ref, bf_ref, an_ref, bn_ref, mask_ref,
                  W_atom_ref, b_atom_ref, Wn1_ref, Wn2_ref, bnei_ref,
                  aw1_ref, aw2_ref, ab_ref, atW_ref, atb_ref,
                  gih_ref, ghh_ref, gbih_ref, gbhh_ref,
                  mw1_ref, mw2_ref, mab_ref, matW_ref, matb_ref,
                  mgih_ref, mghh_ref, mgbih_ref, mgbhh_ref,
                  dW1_ref, db1_ref, dW2_ref, db2_ref, dw3_ref, db3_ref,
                  out_ref):
    af = af_ref[...].reshape(R, ATOM_F)
    bf = bf_ref[...].reshape(MB * M, BOND_F)
    an = an_ref[...]
    bn = bn_ref[...]

    afp = jax.nn.leaky_relu(
        jnp.dot(af, W_atom_ref[...], preferred_element_type=f32) + b_atom_ref[...])
    apre = jnp.dot(af, Wn1_ref[...], preferred_element_type=f32)
    bpre = jnp.dot(bf, Wn2_ref[...], preferred_element_type=f32)

    iota_a = lax.broadcasted_iota(jnp.int32, (A, D, A), 2)
    iota_b = lax.broadcasted_iota(jnp.int32, (A, D, M), 2)
    oh_a = [(an[m][:, :, None] == iota_a).astype(f32) for m in range(MB)]
    oh_b = [(bn[m][:, :, None] == iota_b).astype(f32) for m in range(MB)]

    negm = jnp.where(an == A - 1, NEG, 0.0).astype(f32)
    attm = (an != A - 1).astype(f32)

    afp3 = afp.reshape(MB, A, FP)
    c13 = jnp.sum(afp3 * aw1_ref[0][None], axis=2, keepdims=True)
    nfp3_list = []
    s2e_list = []
    for m in range(MB):
        ga = jnp.dot(oh_a[m].reshape(A * D, A), apre[m * A:(m + 1) * A],
                     preferred_element_type=f32)
        gb = jnp.dot(oh_b[m].reshape(A * D, M), bpre[m * M:(m + 1) * M],
                     preferred_element_type=f32)
        nfp3 = jax.nn.leaky_relu(ga + gb + bnei_ref[...]).reshape(A, D, FP)
        nfp3_list.append(nfp3)
        s2e_list.append(jnp.sum(nfp3 * aw2_ref[0][None], axis=2))
    s2e = jnp.stack(s2e_list, axis=0)
    score = jax.nn.leaky_relu(c13 + s2e + ab_ref[0]) + negm
    attw = _softmax_last(score) * attm
    ctx_pre = jnp.concatenate(
        [jnp.sum(attw[m][:, :, None] * nfp3_list[m], axis=1) for m in range(MB)],
        axis=0)
    sw = jnp.sum(attw, axis=2, keepdims=True).reshape(R, 1)
    ctx = _elu(jnp.dot(ctx_pre, atW_ref[0], preferred_element_type=f32)
                     + sw * atb_ref[0])
    afp = _gru(ctx, afp, gih_ref[0], ghh_ref[0], gbih_ref[0], gbhh_ref[0])

    for i in (1, 2):
        afp3 = afp.reshape(MB, A, FP)
        c13 = jnp.sum(afp3 * aw1_ref[i][None], axis=2, keepdims=True)
        s2row = jnp.sum(afp3 * aw2_ref[i][None], axis=2)
        s2g = jnp.stack(
            [jnp.sum(oh_a[m] * s2row[m:m + 1][:, None, :], axis=2)
             for m in range(MB)], axis=0)
        score = jax.nn.leaky_relu(c13 + s2g + ab_ref[i]) + negm
        attw = _softmax_last(score) * attm
        ctx_pre = jnp.concatenate(
            [jnp.dot(jnp.sum(attw[m][:, :, None] * oh_a[m], axis=1),
                     afp[m * A:(m + 1) * A], preferred_element_type=f32)
             for m in range(MB)], axis=0)
        sw = jnp.sum(attw, axis=2, keepdims=True).reshape(R, 1)
        ctx = _elu(jnp.dot(ctx_pre, atW_ref[i], preferred_element_type=f32)
                         + sw * atb_ref[i])
        afp = _gru(ctx, afp, gih_ref[i], ghh_ref[i], gbih_ref[i], gbhh_ref[i])

    mask2 = mask_ref[...]
    afp3 = afp.reshape(MB, A, FP)
    super_ = jnp.sum(afp3 * mask2[:, :, None], axis=1)
    molneg = jnp.where(mask2 == 0.0, NEG, 0.0).astype(f32)
    act = super_
    for _ in range(2):
        sdot = jnp.sum(super_ * mw1_ref[...], axis=1, keepdims=True)
        adots = jnp.sum(afp3 * mw2_ref[...][None], axis=2)
        score = jax.nn.leaky_relu(sdot + adots + mab_ref[...]) + molneg
        attw = _softmax_last(score) * mask2
        ctxp = jnp.sum(attw[:, :, None] * afp3, axis=1)
        sw = jnp.sum(attw, axis=1, keepdims=True)
        ctx = _elu(jnp.dot(ctxp, matW_ref[...], preferred_element_type=f32)
                         + sw * matb_ref[...])
        super_ = _gru(ctx, super_, mgih_ref[...], mghh_ref[...],
                      mgbih_ref[...], mgbhh_ref[...])
        act = jax.nn.relu(super_)

    h1 = jax.nn.relu(jnp.dot(act, dW1_ref[...], preferred_element_type=f32)
                     + db1_ref[...])
    h2 = jax.nn.relu(jnp.dot(h1, dW2_ref[...], preferred_element_type=f32)
                     + db2_ref[...])
    out_ref[...] = jnp.sum(h2 * dw3_ref[...], axis=1, keepdims=True) + db3_ref[...]


def kernel(atom_features, bond_features, atom_neighbor_list, bond_neighbor_list,
           atom_mask, W_atom, b_atom, W_nei, b_nei, align_W, align_b,
           attend_W, attend_b, gru_Wih, gru_Whh, gru_bih, gru_bhh,
           mol_align_W, mol_align_b, mol_attend_W, mol_attend_b,
           mol_gru_Wih, mol_gru_Whh, mol_gru_bih, mol_gru_bhh,
           dnn_W1, dnn_b1, dnn_W2, dnn_b2, dnn_W3, dnn_b3):
    Wn1 = W_nei[:ATOM_F]
    Wn2 = W_nei[ATOM_F:]
    aw1 = align_W[:, :FP, 0].reshape(3, 1, FP)
    aw2 = align_W[:, FP:, 0].reshape(3, 1, FP)
    ab = align_b.reshape(3, 1, 1)
    atb = attend_b.reshape(3, 1, FP)
    gihT = jnp.swapaxes(gru_Wih, 1, 2)
    ghhT = jnp.swapaxes(gru_Whh, 1, 2)
    gbih = gru_bih.reshape(3, 1, 3 * FP)
    gbhh = gru_bhh.reshape(3, 1, 3 * FP)
    mw1 = mol_align_W[:FP, 0].reshape(1, FP)
    mw2 = mol_align_W[FP:, 0].reshape(1, FP)
    mab = mol_align_b.reshape(1, 1)
    matb = mol_attend_b.reshape(1, FP)
    mgihT = mol_gru_Wih.T
    mghhT = mol_gru_Whh.T
    mgbih = mol_gru_bih.reshape(1, 3 * FP)
    mgbhh = mol_gru_bhh.reshape(1, 3 * FP)
    db1 = dnn_b1.reshape(1, 512)
    db2 = dnn_b2.reshape(1, 128)
    dw3 = dnn_W3.reshape(1, 128)
    db3 = dnn_b3.reshape(1, 1)
    b_atom2 = b_atom.reshape(1, FP)
    b_nei2 = b_nei.reshape(1, FP)

    an32 = atom_neighbor_list.astype(jnp.int32)
    bn32 = bond_neighbor_list.astype(jnp.int32)

    def blk(shape, imap):
        return pl.BlockSpec(shape, imap)

    full = lambda arr: pl.BlockSpec(arr.shape, lambda i: (0,) * arr.ndim)

    in_specs = [
        blk((MB, A, ATOM_F), lambda i: (i, 0, 0)),
        blk((MB, M, BOND_F), lambda i: (i, 0, 0)),
        blk((MB, A, D), lambda i: (i, 0, 0)),
        blk((MB, A, D), lambda i: (i, 0, 0)),
        blk((MB, A), lambda i: (i, 0)),
    ]
    weights = [W_atom, b_atom2, Wn1, Wn2, b_nei2,
               aw1, aw2, ab, attend_W, atb,
               gihT, ghhT, gbih, gbhh,
               mw1, mw2, mab, mol_attend_W, matb,
               mgihT, mghhT, mgbih, mgbhh,
               dnn_W1, db1, dnn_W2, db2, dw3, db3]
    in_specs += [full(w) for w in weights]

    out = pl.pallas_call(
        _fused_kernel,
        grid=(GRID,),
        in_specs=in_specs,
        out_specs=pl.BlockSpec((MB, 1), lambda i: (i, 0)),
        out_shape=jax.ShapeDtypeStruct((B, 1), f32),
        compiler_params=pltpu.CompilerParams(
            dimension_semantics=("arbitrary",)),
    )(atom_features, bond_features, an32, bn32, atom_mask, *weights)
    return out
